# baseline (speedup 1.0000x reference)
import functools

import jax
import jax.numpy as jnp
from jax.experimental import pallas as pl
from jax.experimental.pallas import tpu as pltpu

_CDIM = 64
_BT = 256


def _topk_mask(bits, bt, hdim):
    ones = jnp.full((hdim, 1), 1.0, jnp.float32)

    def count(sel):
        ge = jnp.where(sel, 1.0, 0.0)
        return jnp.dot(ge, ones, preferred_element_type=jnp.float32)

    lo = jnp.zeros((bt, 1), jnp.int32)
    hi = jnp.full((bt, 1), 0x7F000000, jnp.int32)

    def body(_, c):
        lo, hi = c
        mid = lo + ((hi - lo + 1) >> 1)
        pred = count(bits >= mid) >= float(_CDIM)
        return jnp.where(pred, mid, lo), jnp.where(pred, hi, mid - 1)

    lo, hi = jax.lax.fori_loop(0, 31, body, (lo, hi))
    th = lo

    gt = bits > th
    eq = bits == th
    n_gt = count(gt)
    m = float(_CDIM) - n_gt

    iota = jax.lax.broadcasted_iota(jnp.int32, (bt, hdim), 1)
    lo2 = jnp.zeros((bt, 1), jnp.int32)
    hi2 = jnp.full((bt, 1), hdim - 1, jnp.int32)

    def body2(_, c):
        lo2, hi2 = c
        mid = (lo2 + hi2) >> 1
        pred = count(eq & (iota <= mid)) >= m
        return jnp.where(pred, lo2, mid + 1), jnp.where(pred, mid, hi2)

    lo2, hi2 = jax.lax.fori_loop(0, 11, body2, (lo2, hi2))
    mask_eq = eq & (iota <= hi2) & (m > 0)
    return gt | mask_eq


def _block_kernel(x_ref, y_ref, theta_ref, enc_w_ref, enc_b_ref, dec_w_ref,
                  dec_b_ref, out_ref, cpos_ref, cmask_ref, *, bt, hdim):
    j = pl.program_id(1)

    @pl.when(j == 0)
    def _init():
        cpos_ref[0] = -1
        cmask_ref[:, :] = jnp.zeros_like(cmask_ref)

    t0 = j * bt

    x = x_ref[0]
    h = jnp.dot(x, enc_w_ref[:, :], preferred_element_type=jnp.float32)
    h = h + enc_b_ref[0, :][None, :]

    e = h * h
    bits = jax.lax.bitcast_convert_type(e, jnp.int32)
    own = _topk_mask(bits, bt, hdim).astype(jnp.float32)

    theta = theta_ref[0, 0]
    move = jnp.abs(theta - 127) > 64
    it = jax.lax.broadcasted_iota(jnp.int32, (bt, bt), 0)
    isx = jax.lax.broadcasted_iota(jnp.int32, (bt, bt), 1)
    pos_row = jnp.where(move, t0 + jax.lax.broadcasted_iota(
        jnp.int32, (1, bt), 1), -1)
    m2 = jnp.where(isx <= it, jnp.broadcast_to(pos_row, (bt, bt)), -1)
    pm = jnp.max(m2, axis=1, keepdims=True)
    pm = jnp.maximum(pm, cpos_ref[0])
    gather_pos = jnp.maximum(pm, 0)
    srel = gather_pos - t0
    in_blk = srel >= 0
    perm = ((isx == srel) & in_blk).astype(jnp.float32)
    held = jnp.dot(perm, own, preferred_element_type=jnp.float32)
    held = held + (1.0 - in_blk.astype(jnp.float32)) * cmask_ref[0, :][None, :]

    cpos_ref[0] = jnp.max(pm)
    cmask_ref[:, :] = held[bt - 1:bt, :]

    hm = h * held
    yb = jnp.dot(hm, dec_w_ref[:, :], preferred_element_type=jnp.float32)
    yb = yb + dec_b_ref[0, :][None, :]
    yblk = y_ref[0]
    out_ref[0] = jnp.where(yblk == 0.0, 0.0, yb)


@jax.jit
def kernel(x, y, theta, enc_w, enc_b, dec_w, dec_b):
    b, t, idim = x.shape
    hdim = enc_w.shape[1]
    odim = dec_w.shape[1]
    bt = _BT
    nt = t // bt

    theta4 = theta.astype(jnp.int32).reshape(b, nt, 1, bt)
    enc_b2 = enc_b.reshape(1, hdim)
    dec_b2 = dec_b.reshape(1, odim)

    grid = (b, nt)
    out = pl.pallas_call(
        functools.partial(_block_kernel, bt=bt, hdim=hdim),
        grid=grid,
        in_specs=[
            pl.BlockSpec((1, bt, idim), lambda i, j: (i, j, 0)),
            pl.BlockSpec((1, bt, odim), lambda i, j: (i, j, 0)),
            pl.BlockSpec((1, 1, 1, bt), lambda i, j: (i, j, 0, 0)),
            pl.BlockSpec((idim, hdim), lambda i, j: (0, 0)),
            pl.BlockSpec((1, hdim), lambda i, j: (0, 0)),
            pl.BlockSpec((hdim, odim), lambda i, j: (0, 0)),
            pl.BlockSpec((1, odim), lambda i, j: (0, 0)),
        ],
        out_specs=pl.BlockSpec((1, bt, odim), lambda i, j: (i, j, 0)),
        out_shape=jax.ShapeDtypeStruct((b, t, odim), jnp.float32),
        scratch_shapes=[
            pltpu.SMEM((1,), jnp.int32),
            pltpu.VMEM((1, hdim), jnp.float32),
        ],
        compiler_params=pltpu.CompilerParams(
            dimension_semantics=("arbitrary", "arbitrary"),
        ),
    )(x, y, theta4, enc_w, enc_b2, dec_w, dec_b2)
    return out


# speedup vs baseline: 1.5302x; 1.5302x over previous
import functools

import jax
import jax.numpy as jnp
from jax.experimental import pallas as pl
from jax.experimental.pallas import tpu as pltpu

_CDIM = 64
_BT = 256


def _topk_mask(bits, bt, hdim):
    khi = bits >> 16
    lo = jnp.zeros((bt, 1), jnp.int32)
    hi = jnp.full((bt, 1), 0x7F7F, jnp.int32)

    def body(_, c):
        lo, hi = c
        mid = lo + ((hi - lo + 1) >> 1)
        cnt = jnp.sum((khi >= mid).astype(jnp.int32), axis=1, keepdims=True)
        pred = cnt >= _CDIM
        return jnp.where(pred, mid, lo), jnp.where(pred, hi, mid - 1)

    lo, hi = jax.lax.fori_loop(0, 15, body, (lo, hi))
    th_hi = lo

    gt = khi > th_hi
    band = khi == th_hi
    n_gt = jnp.sum(gt.astype(jnp.int32), axis=1, keepdims=True)
    m_rem = _CDIM - n_gt

    iota = jax.lax.broadcasted_iota(jnp.int32, (bt, hdim), 1)
    ckey = jnp.where(band,
                     ((bits & 0xFFFF) << 11) | ((hdim - 1) - iota),
                     -1)

    def wcond(c):
        _, _, m_rem = c
        return jnp.max(m_rem) > 0

    def wbody(c):
        sel, ckey, m_rem = c
        need = m_rem > 0
        mx = jnp.max(ckey, axis=1, keepdims=True)
        pick = (ckey == mx) & need
        sel = jnp.where(pick, 1, sel)
        ckey = jnp.where(pick, -1, ckey)
        return sel, ckey, m_rem - need.astype(jnp.int32)

    sel, _, _ = jax.lax.while_loop(
        wcond, wbody, (jnp.zeros((bt, hdim), jnp.int32), ckey, m_rem))
    return gt | (sel > 0)


def _block_kernel(x_ref, y_ref, theta_ref, enc_w_ref, enc_b_ref, dec_w_ref,
                  dec_b_ref, out_ref, cpos_ref, cmask_ref, *, bt, hdim):
    j = pl.program_id(1)

    @pl.when(j == 0)
    def _init():
        cpos_ref[0] = -1
        cmask_ref[:, :] = jnp.zeros_like(cmask_ref)

    t0 = j * bt

    x = x_ref[0]
    h = jnp.dot(x, enc_w_ref[:, :], preferred_element_type=jnp.float32)
    h = h + enc_b_ref[0, :][None, :]

    e = h * h
    bits = jax.lax.bitcast_convert_type(e, jnp.int32)
    own = _topk_mask(bits, bt, hdim).astype(jnp.float32)

    theta = theta_ref[0, 0]
    move = jnp.abs(theta - 127) > 64
    it = jax.lax.broadcasted_iota(jnp.int32, (bt, bt), 0)
    isx = jax.lax.broadcasted_iota(jnp.int32, (bt, bt), 1)
    pos_row = jnp.where(move, t0 + jax.lax.broadcasted_iota(
        jnp.int32, (1, bt), 1), -1)
    m2 = jnp.where(isx <= it, jnp.broadcast_to(pos_row, (bt, bt)), -1)
    pm = jnp.max(m2, axis=1, keepdims=True)
    pm = jnp.maximum(pm, cpos_ref[0])
    gather_pos = jnp.maximum(pm, 0)
    srel = gather_pos - t0
    in_blk = srel >= 0
    perm = ((isx == srel) & in_blk).astype(jnp.float32)
    held = jnp.dot(perm, own, preferred_element_type=jnp.float32)
    held = held + (1.0 - in_blk.astype(jnp.float32)) * cmask_ref[0, :][None, :]

    cpos_ref[0] = jnp.max(pm)
    cmask_ref[:, :] = held[bt - 1:bt, :]

    hm = h * held
    yb = jnp.dot(hm, dec_w_ref[:, :], preferred_element_type=jnp.float32)
    yb = yb + dec_b_ref[0, :][None, :]
    yblk = y_ref[0]
    out_ref[0] = jnp.where(yblk == 0.0, 0.0, yb)


@jax.jit
def kernel(x, y, theta, enc_w, enc_b, dec_w, dec_b):
    b, t, idim = x.shape
    hdim = enc_w.shape[1]
    odim = dec_w.shape[1]
    bt = _BT
    nt = t // bt

    theta4 = theta.astype(jnp.int32).reshape(b, nt, 1, bt)
    enc_b2 = enc_b.reshape(1, hdim)
    dec_b2 = dec_b.reshape(1, odim)

    grid = (b, nt)
    out = pl.pallas_call(
        functools.partial(_block_kernel, bt=bt, hdim=hdim),
        grid=grid,
        in_specs=[
            pl.BlockSpec((1, bt, idim), lambda i, j: (i, j, 0)),
            pl.BlockSpec((1, bt, odim), lambda i, j: (i, j, 0)),
            pl.BlockSpec((1, 1, 1, bt), lambda i, j: (i, j, 0, 0)),
            pl.BlockSpec((idim, hdim), lambda i, j: (0, 0)),
            pl.BlockSpec((1, hdim), lambda i, j: (0, 0)),
            pl.BlockSpec((hdim, odim), lambda i, j: (0, 0)),
            pl.BlockSpec((1, odim), lambda i, j: (0, 0)),
        ],
        out_specs=pl.BlockSpec((1, bt, odim), lambda i, j: (i, j, 0)),
        out_shape=jax.ShapeDtypeStruct((b, t, odim), jnp.float32),
        scratch_shapes=[
            pltpu.SMEM((1,), jnp.int32),
            pltpu.VMEM((1, hdim), jnp.float32),
        ],
        compiler_params=pltpu.CompilerParams(
            dimension_semantics=("arbitrary", "arbitrary"),
        ),
    )(x, y, theta4, enc_w, enc_b2, dec_w, dec_b2)
    return out
